# Initial kernel scaffold; baseline (speedup 1.0000x reference)
import jax
import jax.numpy as jnp
from jax.experimental import pallas as pl


def kernel(loc, conf, priors):
    raise NotImplementedError("write your pallas kernel here")


# speedup vs baseline: 10.8167x; 10.8167x over previous
import jax
import jax.numpy as jnp
from jax import lax
from jax.experimental import pallas as pl
from jax.experimental.pallas import tpu as pltpu

_NUM_CLASSES = 21
_CM1 = _NUM_CLASSES - 1
_TOP_K = 200
_CONF_THRESH = 0.01
_NMS_THRESH = 0.45
_VAR_XY, _VAR_WH = 0.1, 0.2

_N = 8732
_NPAD = 8960
_NBLK = 896


def _softmax_kernel(conf_ref, out_ref):
    c = conf_ref[0]
    m = jnp.max(c, axis=0, keepdims=True)
    e = jnp.exp(c - m)
    s = e / jnp.sum(e, axis=0, keepdims=True)
    pos = pl.program_id(1) * _NBLK + lax.broadcasted_iota(
        jnp.int32, (_CM1, _NBLK), 1)
    out_ref[0] = jnp.where(pos < _N, s[1:, :], 0.0)


def _nms_kernel(s_ref, loc_ref, pri_ref, outs_ref, outb_ref, iou_ref):
    s = s_ref[0]
    lx = loc_ref[0, 0]
    ly = loc_ref[0, 1]
    lw = loc_ref[0, 2]
    lh = loc_ref[0, 3]
    px = pri_ref[0, 0]
    py = pri_ref[0, 1]
    pw = pri_ref[0, 2]
    ph = pri_ref[0, 3]

    cx = px + lx * _VAR_XY * pw
    cy = py + ly * _VAR_XY * ph
    w = pw * jnp.exp(lw * _VAR_WH)
    h = ph * jnp.exp(lh * _VAR_WH)
    x1 = cx - w * 0.5
    y1 = cy - h * 0.5
    x2 = cx + w * 0.5
    y2 = cy + h * 0.5
    area = (x2 - x1) * (y2 - y1)

    for r in range(_TOP_K // 8):
        sl = slice(r * 8, r * 8 + 8)
        ix1 = jnp.maximum(x1[sl][:, None, :], x1[None, :, :])
        iy1 = jnp.maximum(y1[sl][:, None, :], y1[None, :, :])
        ix2 = jnp.minimum(x2[sl][:, None, :], x2[None, :, :])
        iy2 = jnp.minimum(y2[sl][:, None, :], y2[None, :, :])
        inter = jnp.maximum(ix2 - ix1, 0.0) * jnp.maximum(iy2 - iy1, 0.0)
        union = area[sl][:, None, :] + area[None, :, :] - inter
        iou_ref[sl] = inter / jnp.maximum(union, 1e-12)

    row_iota = lax.broadcasted_iota(jnp.int32, (_TOP_K, 128), 0)

    def body(i, sup):
        rs = row_iota == i
        s_i = jnp.max(jnp.where(rs, s, 0.0), axis=0, keepdims=True)
        sup_i = jnp.max(jnp.where(rs, sup, 0.0), axis=0, keepdims=True)
        kept_i = (sup_i == 0.0) & (s_i > _CONF_THRESH)
        row = iou_ref[pl.ds(i, 1)][0]
        hit = (row > _NMS_THRESH) & (row_iota > i) & kept_i
        return jnp.maximum(sup, jnp.where(hit, 1.0, 0.0))

    sup = lax.fori_loop(0, _TOP_K, body, jnp.zeros((_TOP_K, 128), jnp.float32))
    m = jnp.where((sup == 0.0) & (s > _CONF_THRESH), 1.0, 0.0)
    outs_ref[0] = s * m
    outb_ref[0, 0] = x1 * m
    outb_ref[0, 1] = y1 * m
    outb_ref[0, 2] = x2 * m
    outb_ref[0, 3] = y2 * m


def kernel(loc, conf, priors):
    B, N, C = conf.shape
    K = _TOP_K

    conf_t = jnp.pad(jnp.swapaxes(conf, 1, 2),
                     ((0, 0), (0, 0), (0, _NPAD - N)))

    scores = pl.pallas_call(
        _softmax_kernel,
        grid=(B, _NPAD // _NBLK),
        in_specs=[pl.BlockSpec((1, C, _NBLK), lambda b, n: (b, 0, n))],
        out_specs=pl.BlockSpec((1, _CM1, _NBLK), lambda b, n: (b, 0, n)),
        out_shape=jax.ShapeDtypeStruct((B, _CM1, _NPAD), jnp.float32),
        compiler_params=pltpu.CompilerParams(
            dimension_semantics=("parallel", "arbitrary")),
    )(conf_t)

    top_s, top_i = lax.top_k(scores, K)
    top_i = jnp.minimum(top_i, N - 1)

    loc_g = jnp.take_along_axis(loc[:, None], top_i[..., None], axis=2)
    pri_g = jnp.take(priors, top_i, axis=0)

    s_in = jnp.transpose(top_s, (1, 2, 0))
    loc_in = jnp.transpose(loc_g, (1, 3, 2, 0))
    pri_in = jnp.transpose(pri_g, (1, 3, 2, 0))

    outs, outb = pl.pallas_call(
        _nms_kernel,
        grid=(_CM1,),
        in_specs=[
            pl.BlockSpec((1, K, B), lambda c: (c, 0, 0)),
            pl.BlockSpec((1, 4, K, B), lambda c: (c, 0, 0, 0)),
            pl.BlockSpec((1, 4, K, B), lambda c: (c, 0, 0, 0)),
        ],
        out_specs=[
            pl.BlockSpec((1, K, B), lambda c: (c, 0, 0)),
            pl.BlockSpec((1, 4, K, B), lambda c: (c, 0, 0, 0)),
        ],
        out_shape=[
            jax.ShapeDtypeStruct((_CM1, K, B), jnp.float32),
            jax.ShapeDtypeStruct((_CM1, 4, K, B), jnp.float32),
        ],
        scratch_shapes=[pltpu.VMEM((K, K, B), jnp.float32)],
        compiler_params=pltpu.CompilerParams(
            dimension_semantics=("parallel",),
            vmem_limit_bytes=56 * 1024 * 1024),
    )(s_in, loc_in, pri_in)

    out_s = jnp.transpose(outs, (2, 0, 1))
    out_b = jnp.transpose(outb, (3, 0, 2, 1))
    det = jnp.concatenate([out_s[..., None], out_b], axis=-1)
    bg = jnp.zeros((B, 1, K, 5), det.dtype)
    return jnp.concatenate([bg, det], axis=1)


# speedup vs baseline: 13.6237x; 1.2595x over previous
import jax
import jax.numpy as jnp
from jax import lax
from jax.experimental import pallas as pl
from jax.experimental.pallas import tpu as pltpu

_NUM_CLASSES = 21
_CM1 = _NUM_CLASSES - 1
_TOP_K = 200
_CONF_THRESH = 0.01
_NMS_THRESH = 0.45
_VAR_XY, _VAR_WH = 0.1, 0.2

_N = 8732
_NPAD = 8960
_NBLK = 896


def _softmax_kernel(conf_ref, out_ref):
    c = conf_ref[0]
    m = jnp.max(c, axis=0, keepdims=True)
    e = jnp.exp(c - m)
    s = e / jnp.sum(e, axis=0, keepdims=True)
    pos = pl.program_id(1) * _NBLK + lax.broadcasted_iota(
        jnp.int32, (_CM1, _NBLK), 1)
    out_ref[0] = jnp.where(pos < _N, s[1:, :], 0.0)


def _nms_kernel(s_ref, loc_ref, pri_ref, outs_ref, outb_ref, iou_ref):
    s = s_ref[0]
    lx = loc_ref[0, 0]
    ly = loc_ref[0, 1]
    lw = loc_ref[0, 2]
    lh = loc_ref[0, 3]
    px = pri_ref[0, 0]
    py = pri_ref[0, 1]
    pw = pri_ref[0, 2]
    ph = pri_ref[0, 3]

    cx = px + lx * _VAR_XY * pw
    cy = py + ly * _VAR_XY * ph
    w = pw * jnp.exp(lw * _VAR_WH)
    h = ph * jnp.exp(lh * _VAR_WH)
    x1 = cx - w * 0.5
    y1 = cy - h * 0.5
    x2 = cx + w * 0.5
    y2 = cy + h * 0.5
    area = (x2 - x1) * (y2 - y1)

    for r in range(_TOP_K // 8):
        sl = slice(r * 8, r * 8 + 8)
        ix1 = jnp.maximum(x1[sl][:, None, :], x1[None, :, :])
        iy1 = jnp.maximum(y1[sl][:, None, :], y1[None, :, :])
        ix2 = jnp.minimum(x2[sl][:, None, :], x2[None, :, :])
        iy2 = jnp.minimum(y2[sl][:, None, :], y2[None, :, :])
        inter = jnp.maximum(ix2 - ix1, 0.0) * jnp.maximum(iy2 - iy1, 0.0)
        union = area[sl][:, None, :] + area[None, :, :] - inter
        iou_ref[sl] = inter / jnp.maximum(union, 1e-12)

    row_iota = lax.broadcasted_iota(jnp.int32, (_TOP_K, 128), 0)

    def body(i, sup):
        rs = row_iota == i
        s_i = jnp.max(jnp.where(rs, s, 0.0), axis=0, keepdims=True)
        sup_i = jnp.max(jnp.where(rs, sup, 0.0), axis=0, keepdims=True)
        kept_i = (sup_i == 0.0) & (s_i > _CONF_THRESH)
        row = iou_ref[pl.ds(i, 1)][0]
        hit = (row > _NMS_THRESH) & (row_iota > i) & kept_i
        return jnp.maximum(sup, jnp.where(hit, 1.0, 0.0))

    sup = lax.fori_loop(0, _TOP_K, body, jnp.zeros((_TOP_K, 128), jnp.float32))
    m = jnp.where((sup == 0.0) & (s > _CONF_THRESH), 1.0, 0.0)
    outs_ref[0] = s * m
    outb_ref[0, 0] = x1 * m
    outb_ref[0, 1] = y1 * m
    outb_ref[0, 2] = x2 * m
    outb_ref[0, 3] = y2 * m


def kernel(loc, conf, priors):
    B, N, C = conf.shape
    K = _TOP_K

    conf_t = jnp.pad(jnp.swapaxes(conf, 1, 2),
                     ((0, 0), (0, 0), (0, _NPAD - N)))

    scores = pl.pallas_call(
        _softmax_kernel,
        grid=(B, _NPAD // _NBLK),
        in_specs=[pl.BlockSpec((1, C, _NBLK), lambda b, n: (b, 0, n))],
        out_specs=pl.BlockSpec((1, _CM1, _NBLK), lambda b, n: (b, 0, n)),
        out_shape=jax.ShapeDtypeStruct((B, _CM1, _NPAD), jnp.float32),
        compiler_params=pltpu.CompilerParams(
            dimension_semantics=("parallel", "arbitrary")),
    )(conf_t)

    top_s, top_i = lax.approx_max_k(scores, K, recall_target=1.0)
    top_i = jnp.minimum(top_i, N - 1)

    loc_g = jnp.take_along_axis(loc[:, None], top_i[..., None], axis=2)
    pri_g = jnp.take(priors, top_i, axis=0)

    s_in = jnp.transpose(top_s, (1, 2, 0))
    loc_in = jnp.transpose(loc_g, (1, 3, 2, 0))
    pri_in = jnp.transpose(pri_g, (1, 3, 2, 0))

    outs, outb = pl.pallas_call(
        _nms_kernel,
        grid=(_CM1,),
        in_specs=[
            pl.BlockSpec((1, K, B), lambda c: (c, 0, 0)),
            pl.BlockSpec((1, 4, K, B), lambda c: (c, 0, 0, 0)),
            pl.BlockSpec((1, 4, K, B), lambda c: (c, 0, 0, 0)),
        ],
        out_specs=[
            pl.BlockSpec((1, K, B), lambda c: (c, 0, 0)),
            pl.BlockSpec((1, 4, K, B), lambda c: (c, 0, 0, 0)),
        ],
        out_shape=[
            jax.ShapeDtypeStruct((_CM1, K, B), jnp.float32),
            jax.ShapeDtypeStruct((_CM1, 4, K, B), jnp.float32),
        ],
        scratch_shapes=[pltpu.VMEM((K, K, B), jnp.float32)],
        compiler_params=pltpu.CompilerParams(
            dimension_semantics=("parallel",),
            vmem_limit_bytes=56 * 1024 * 1024),
    )(s_in, loc_in, pri_in)

    out_s = jnp.transpose(outs, (2, 0, 1))
    out_b = jnp.transpose(outb, (3, 0, 2, 1))
    det = jnp.concatenate([out_s[..., None], out_b], axis=-1)
    bg = jnp.zeros((B, 1, K, 5), det.dtype)
    return jnp.concatenate([bg, det], axis=1)


# speedup vs baseline: 25.7466x; 1.8898x over previous
import jax
import jax.numpy as jnp
from jax import lax
from jax.experimental import pallas as pl
from jax.experimental.pallas import tpu as pltpu

_NUM_CLASSES = 21
_CM1 = _NUM_CLASSES - 1
_TOP_K = 200
_CONF_THRESH = 0.01
_NMS_THRESH = 0.45
_VAR_XY, _VAR_WH = 0.1, 0.2

_N = 8732
_NPAD = 8960
_NBLK = 896


def _topk_kernel(conf_ref, outs_ref, outi_ref, s_scr):
    c = conf_ref[0]
    m = jnp.max(c, axis=0, keepdims=True)
    e = jnp.exp(c - m)
    sm = e / jnp.sum(e, axis=0, keepdims=True)
    pos = lax.broadcasted_iota(jnp.int32, (_CM1, _NPAD), 1)
    s_scr[...] = jnp.where(pos < _N, sm[1:, :], -1.0)
    lane = lax.broadcasted_iota(jnp.int32, (_CM1, _TOP_K), 1)

    def body(k, carry):
        acc_s, acc_i = carry
        s = s_scr[...]
        mx = jnp.max(s, axis=1, keepdims=True)
        ix = jnp.argmax(s, axis=1, keepdims=True)
        acc_s = jnp.where(lane == k, mx, acc_s)
        acc_i = jnp.where(lane == k, ix, acc_i)
        s_scr[...] = jnp.where(pos == ix, -1.0, s)
        return acc_s, acc_i

    acc_s, acc_i = lax.fori_loop(
        0, _TOP_K, body,
        (jnp.zeros((_CM1, _TOP_K), jnp.float32),
         jnp.zeros((_CM1, _TOP_K), jnp.int32)))
    outs_ref[0] = acc_s
    outi_ref[0] = acc_i


def _nms_kernel(s_ref, loc_ref, pri_ref, outs_ref, outb_ref, iou_ref):
    s = s_ref[0]
    lx = loc_ref[0, 0]
    ly = loc_ref[0, 1]
    lw = loc_ref[0, 2]
    lh = loc_ref[0, 3]
    px = pri_ref[0, 0]
    py = pri_ref[0, 1]
    pw = pri_ref[0, 2]
    ph = pri_ref[0, 3]

    cx = px + lx * _VAR_XY * pw
    cy = py + ly * _VAR_XY * ph
    w = pw * jnp.exp(lw * _VAR_WH)
    h = ph * jnp.exp(lh * _VAR_WH)
    x1 = cx - w * 0.5
    y1 = cy - h * 0.5
    x2 = cx + w * 0.5
    y2 = cy + h * 0.5
    area = (x2 - x1) * (y2 - y1)

    for r in range(_TOP_K // 8):
        sl = slice(r * 8, r * 8 + 8)
        ix1 = jnp.maximum(x1[sl][:, None, :], x1[None, :, :])
        iy1 = jnp.maximum(y1[sl][:, None, :], y1[None, :, :])
        ix2 = jnp.minimum(x2[sl][:, None, :], x2[None, :, :])
        iy2 = jnp.minimum(y2[sl][:, None, :], y2[None, :, :])
        inter = jnp.maximum(ix2 - ix1, 0.0) * jnp.maximum(iy2 - iy1, 0.0)
        union = area[sl][:, None, :] + area[None, :, :] - inter
        iou_ref[sl] = inter / jnp.maximum(union, 1e-12)

    row_iota = lax.broadcasted_iota(jnp.int32, (_TOP_K, 128), 0)

    def body(i, sup):
        rs = row_iota == i
        s_i = jnp.max(jnp.where(rs, s, 0.0), axis=0, keepdims=True)
        sup_i = jnp.max(jnp.where(rs, sup, 0.0), axis=0, keepdims=True)
        kept_i = (sup_i == 0.0) & (s_i > _CONF_THRESH)
        row = iou_ref[pl.ds(i, 1)][0]
        hit = (row > _NMS_THRESH) & (row_iota > i) & kept_i
        return jnp.maximum(sup, jnp.where(hit, 1.0, 0.0))

    sup = lax.fori_loop(0, _TOP_K, body, jnp.zeros((_TOP_K, 128), jnp.float32))
    m = jnp.where((sup == 0.0) & (s > _CONF_THRESH), 1.0, 0.0)
    outs_ref[0] = s * m
    outb_ref[0, 0] = x1 * m
    outb_ref[0, 1] = y1 * m
    outb_ref[0, 2] = x2 * m
    outb_ref[0, 3] = y2 * m


def kernel(loc, conf, priors):
    B, N, C = conf.shape
    K = _TOP_K

    conf_t = jnp.pad(jnp.swapaxes(conf, 1, 2),
                     ((0, 0), (0, 0), (0, _NPAD - N)))

    top_s, top_i = pl.pallas_call(
        _topk_kernel,
        grid=(B,),
        in_specs=[pl.BlockSpec((1, C, _NPAD), lambda b: (b, 0, 0))],
        out_specs=[
            pl.BlockSpec((1, _CM1, K), lambda b: (b, 0, 0)),
            pl.BlockSpec((1, _CM1, K), lambda b: (b, 0, 0)),
        ],
        out_shape=[
            jax.ShapeDtypeStruct((B, _CM1, K), jnp.float32),
            jax.ShapeDtypeStruct((B, _CM1, K), jnp.int32),
        ],
        scratch_shapes=[pltpu.VMEM((_CM1, _NPAD), jnp.float32)],
        compiler_params=pltpu.CompilerParams(
            dimension_semantics=("parallel",)),
    )(conf_t)
    top_i = jnp.minimum(top_i, N - 1)

    loc_g = jnp.take_along_axis(loc[:, None], top_i[..., None], axis=2)
    pri_g = jnp.take(priors, top_i, axis=0)

    s_in = jnp.transpose(top_s, (1, 2, 0))
    loc_in = jnp.transpose(loc_g, (1, 3, 2, 0))
    pri_in = jnp.transpose(pri_g, (1, 3, 2, 0))

    outs, outb = pl.pallas_call(
        _nms_kernel,
        grid=(_CM1,),
        in_specs=[
            pl.BlockSpec((1, K, B), lambda c: (c, 0, 0)),
            pl.BlockSpec((1, 4, K, B), lambda c: (c, 0, 0, 0)),
            pl.BlockSpec((1, 4, K, B), lambda c: (c, 0, 0, 0)),
        ],
        out_specs=[
            pl.BlockSpec((1, K, B), lambda c: (c, 0, 0)),
            pl.BlockSpec((1, 4, K, B), lambda c: (c, 0, 0, 0)),
        ],
        out_shape=[
            jax.ShapeDtypeStruct((_CM1, K, B), jnp.float32),
            jax.ShapeDtypeStruct((_CM1, 4, K, B), jnp.float32),
        ],
        scratch_shapes=[pltpu.VMEM((K, K, B), jnp.float32)],
        compiler_params=pltpu.CompilerParams(
            dimension_semantics=("parallel",),
            vmem_limit_bytes=56 * 1024 * 1024),
    )(s_in, loc_in, pri_in)

    out_s = jnp.transpose(outs, (2, 0, 1))
    out_b = jnp.transpose(outb, (3, 0, 2, 1))
    det = jnp.concatenate([out_s[..., None], out_b], axis=-1)
    bg = jnp.zeros((B, 1, K, 5), det.dtype)
    return jnp.concatenate([bg, det], axis=1)
